# Initial kernel scaffold; baseline (speedup 1.0000x reference)
import jax
import jax.numpy as jnp
from jax.experimental import pallas as pl


def kernel(x, w1, b1):
    raise NotImplementedError("write your pallas kernel here")


# speedup vs baseline: 5.3721x; 5.3721x over previous
import functools

import jax
import jax.numpy as jnp
from jax.experimental import pallas as pl
from jax.experimental.pallas import tpu as pltpu

_SQRT_2_OVER_PI = 0.7978845608028654
_GELU_C = 0.044715

_STRIDE = 3
_DIL = 2
_NPH = 3
_NTAP = 9
_UP = 48
_SP = 48
_OWP = 32
_OHP = 32


def _gelu_tail(v1):
    v3 = v1 * v1
    v5 = (v3 * v1) * _GELU_C
    v7 = (v1 + v5) * _SQRT_2_OVER_PI
    v10 = (v1 * 0.5) * (1.0 + jnp.tanh(v7))
    return v10 + v5


def _conv_gelu_body(xs_ref, g_ref, b_ref, o_ref):
    acc = jnp.zeros((_OHP, 128), jnp.float32)
    for p in range(_NTAP):
        lhs = xs_ref[0, pl.ds(_DIL * p, _OHP), :]
        acc += jnp.dot(lhs, g_ref[p], preferred_element_type=jnp.float32)
    o_ref[0] = _gelu_tail(acc + b_ref[...])


@jax.jit
def kernel(x, w1, b1):
    N, Cin, H, W = map(int, x.shape)
    Cout, _, KH, KW = map(int, w1.shape)
    OH = (H - _DIL * (KH - 1) - 1) // _STRIDE + 1
    OW = (W - _DIL * (KW - 1) - 1) // _STRIDE + 1
    Ccomb = Cin * _NPH * _NPH
    LANES = Ccomb * _SP

    hp = _STRIDE * (_UP - 1) + _DIL * (_NPH - 1) + 1 - H
    wp = _STRIDE * (_SP - 1) + _DIL * (_NPH - 1) + 1 - W
    xp = jnp.pad(x, ((0, 0), (0, 0), (0, hp), (0, wp)))
    ph = [[xp[:, :, _DIL * t::_STRIDE, _DIL * r::_STRIDE][:, :, :_UP, :_SP]
           for r in range(_NPH)] for t in range(_NPH)]
    xph = jnp.stack([jnp.stack(row, axis=0) for row in ph], axis=0)
    xs = xph.transpose(2, 4, 3, 0, 1, 5).reshape(N, _UP, LANES)
    xs = xs.astype(jnp.bfloat16)

    w8 = jnp.pad(w1.astype(jnp.float32),
                 ((0, 0), (0, 0), (0, _NPH * _NTAP - KH),
                  (0, _NPH * _NTAP - KW)))
    w9 = w8.reshape(Cout, Cin, _NTAP, _NPH, _NTAP, _NPH)
    s_idx = jnp.arange(_SP)[None, :, None]
    w_idx = jnp.arange(_OWP)[None, None, :]
    q_idx = jnp.arange(_NTAP)[:, None, None]
    sel = ((s_idx == w_idx + _DIL * q_idx) & (w_idx < OW)).astype(jnp.float32)
    g = jnp.einsum('oiptqr,qsw->pitrsow', w9, sel)
    g = g.reshape(_NTAP, LANES, Cout * _OWP).astype(jnp.bfloat16)

    bvec = jnp.repeat(b1.astype(jnp.float32), _OWP).reshape(1, Cout * _OWP)

    cost = pl.CostEstimate(
        flops=2 * N * _NTAP * _OHP * LANES * Cout * _OWP,
        transcendentals=N * _OHP * Cout * _OWP,
        bytes_accessed=xs.size * 2 + g.size * 2 + N * _OHP * Cout * _OWP * 4,
    )
    out2 = pl.pallas_call(
        _conv_gelu_body,
        out_shape=jax.ShapeDtypeStruct((N, _OHP, Cout * _OWP), jnp.float32),
        grid=(N,),
        in_specs=[
            pl.BlockSpec((1, _UP, LANES), lambda n: (n, 0, 0)),
            pl.BlockSpec((_NTAP, LANES, Cout * _OWP), lambda n: (0, 0, 0)),
            pl.BlockSpec((1, Cout * _OWP), lambda n: (0, 0)),
        ],
        out_specs=pl.BlockSpec((1, _OHP, Cout * _OWP), lambda n: (n, 0, 0)),
        compiler_params=pltpu.CompilerParams(
            dimension_semantics=("parallel",),
            vmem_limit_bytes=48 * 1024 * 1024),
        cost_estimate=cost,
    )(xs, g, bvec)

    out = out2.reshape(N, _OHP, Cout, _OWP)[:, :OH, :, :OW]
    return out.transpose(0, 2, 1, 3)


# speedup vs baseline: 20.6820x; 3.8499x over previous
import jax
import jax.numpy as jnp
from jax.experimental import pallas as pl
from jax.experimental.pallas import tpu as pltpu

_SQRT_2_OVER_PI = 0.7978845608028654
_GELU_C = 0.044715

_STRIDE = 3
_DIL = 2
_NPH = 3
_NTAP = 9
_UP = 56
_SP = 48
_OWP = 32
_OHP = 32
_LB = 16 * _SP * _NPH
_T_OF = (0, 2, 1)
_DT = (0, 1, 0)


def _gelu_tail(v1):
    v3 = v1 * v1
    v5 = (v3 * v1) * _GELU_C
    v7 = (v1 + v5) * _SQRT_2_OVER_PI
    v10 = (v1 * 0.5) * (1.0 + jnp.tanh(v7))
    return v10 + v5


def _conv_gelu_body(xs_ref, g_ref, b_ref, o_ref):
    acc = jnp.zeros((_OHP, 128), jnp.float32)
    for p in range(_NTAP):
        for ph in range(_NPH):
            lhs = xs_ref[0, pl.ds(_DIL * p + _DT[ph], _OHP),
                         pl.ds(ph * _LB, _LB)]
            acc += jnp.dot(lhs, g_ref[_NPH * p + ph],
                           preferred_element_type=jnp.float32)
    o_ref[0] = _gelu_tail(acc + b_ref[...])


@jax.jit
def kernel(x, w1, b1):
    N, Cin, H, W = map(int, x.shape)
    Cout, _, KH, KW = map(int, w1.shape)
    OH = (H - _DIL * (KH - 1) - 1) // _STRIDE + 1
    OW = (W - _DIL * (KW - 1) - 1) // _STRIDE + 1

    xp = jnp.pad(x.astype(jnp.bfloat16),
                 ((0, 0), (0, 0), (0, _STRIDE * _UP - H),
                  (0, _STRIDE * _SP - W)))
    xr = xp.reshape(N, Cin, _UP, _NPH, _SP, _NPH)
    xs = xr.transpose(0, 2, 3, 1, 4, 5).reshape(N, _UP, _NPH * _LB)

    w8 = jnp.pad(w1.astype(jnp.float32),
                 ((0, 0), (0, 0), (0, _NPH * _NTAP - KH),
                  (0, _NPH * _NTAP - KW)))
    w9 = w8.reshape(Cout, Cin, _NTAP, _NPH, _NTAP, _NPH)
    perm = jnp.array(_T_OF)
    w10 = jnp.take(jnp.take(w9, perm, axis=3), perm, axis=5)
    s_idx = jnp.arange(_SP)[None, None, :, None]
    w_idx = jnp.arange(_OWP)[None, None, None, :]
    q_idx = jnp.arange(_NTAP)[None, :, None, None]
    dr = jnp.array(_DT)[:, None, None, None]
    sel = ((s_idx == w_idx + _DIL * q_idx + dr)
           & (w_idx < OW)).astype(jnp.float32)
    g = jnp.einsum('oipTqR,Rqsw->pTisRow', w10, sel)
    g = g.reshape(_NPH * _NTAP, _LB, Cout * _OWP).astype(jnp.bfloat16)

    bvec = jnp.repeat(b1.astype(jnp.float32), _OWP).reshape(1, Cout * _OWP)

    cost = pl.CostEstimate(
        flops=2 * N * _NTAP * _NPH * _OHP * _LB * Cout * _OWP,
        transcendentals=N * _OHP * Cout * _OWP,
        bytes_accessed=xs.size * 2 + g.size * 2 + N * _OHP * Cout * _OWP * 4,
    )
    out2 = pl.pallas_call(
        _conv_gelu_body,
        out_shape=jax.ShapeDtypeStruct((N, _OHP, Cout * _OWP), jnp.float32),
        grid=(N,),
        in_specs=[
            pl.BlockSpec((1, _UP, _NPH * _LB), lambda n: (n, 0, 0)),
            pl.BlockSpec((_NPH * _NTAP, _LB, Cout * _OWP), lambda n: (0, 0, 0)),
            pl.BlockSpec((1, Cout * _OWP), lambda n: (0, 0)),
        ],
        out_specs=pl.BlockSpec((1, _OHP, Cout * _OWP), lambda n: (n, 0, 0)),
        compiler_params=pltpu.CompilerParams(
            dimension_semantics=("parallel",),
            vmem_limit_bytes=48 * 1024 * 1024),
        cost_estimate=cost,
    )(xs, g, bvec)

    out = out2.reshape(N, _OHP, Cout, _OWP)[:, :OH, :, :OW]
    return out.transpose(0, 2, 1, 3)


# speedup vs baseline: 41.1098x; 1.9877x over previous
import functools

import jax
import jax.numpy as jnp
from jax.experimental import pallas as pl
from jax.experimental.pallas import tpu as pltpu

_SQRT_2_OVER_PI = 0.7978845608028654
_GELU_C = 0.044715

_STRIDE = 3
_DIL = 2
_NPH = 3
_NTAP = 9
_UP = 56
_OWP = 32
_OHP = 32
_T_OF = (0, 2, 1)
_DT = (0, 1, 0)


def _gelu_tail(v1):
    v3 = v1 * v1
    v5 = (v3 * v1) * _GELU_C
    v7 = (v1 + v5) * _SQRT_2_OVER_PI
    v10 = (v1 * 0.5) * (1.0 + jnp.tanh(v7))
    return v10 + v5


def _conv_gelu_body(xs_ref, g_ref, b_ref, o_ref, *, lb):
    acc = jnp.zeros((_OHP, 128), jnp.float32)
    for p in range(_NTAP):
        for ph in range(_NPH):
            lhs = xs_ref[0, pl.ds(_DIL * p + _DT[ph], _OHP),
                         pl.ds(ph * lb, lb)]
            acc += jnp.dot(lhs, g_ref[_NPH * p + ph],
                           preferred_element_type=jnp.float32)
    o_ref[0] = _gelu_tail(acc + b_ref[...])


@jax.jit
def kernel(x, w1, b1):
    N, Cin, H, W = map(int, x.shape)
    Cout, _, KH, KW = map(int, w1.shape)
    OH = (H - _DIL * (KH - 1) - 1) // _STRIDE + 1
    OW = (W - _DIL * (KW - 1) - 1) // _STRIDE + 1
    LB = Cin * W

    xp = jnp.pad(x.astype(jnp.bfloat16),
                 ((0, 0), (0, 0), (0, _STRIDE * _UP - H), (0, 0)))
    xr = xp.reshape(N, Cin, _UP, _NPH, W)
    xs = xr.transpose(0, 2, 3, 1, 4).reshape(N, _UP, _NPH * LB)

    w8 = jnp.pad(w1.astype(jnp.bfloat16),
                 ((0, 0), (0, 0), (0, _NPH * _NTAP - KH), (0, 0)))
    w9 = w8.reshape(Cout, Cin, _NTAP, _NPH, KW)
    w10 = jnp.take(w9, jnp.array(_T_OF), axis=3)
    c_idx = jnp.arange(W)[:, None, None]
    w_idx = jnp.arange(_OWP)[None, :, None]
    k_idx = jnp.arange(KW)[None, None, :]
    sel = ((c_idx == _STRIDE * w_idx + _DIL * k_idx)
           & (w_idx < OW)).astype(jnp.bfloat16)
    g = jnp.einsum('oipTk,cwk->pTicow', w10, sel,
                   preferred_element_type=jnp.float32)
    g = g.reshape(_NPH * _NTAP, LB, Cout * _OWP).astype(jnp.bfloat16)

    bvec = jnp.repeat(b1.astype(jnp.float32), _OWP).reshape(1, Cout * _OWP)

    cost = pl.CostEstimate(
        flops=2 * N * _NTAP * _NPH * _OHP * LB * Cout * _OWP,
        transcendentals=N * _OHP * Cout * _OWP,
        bytes_accessed=xs.size * 2 + g.size * 2 + N * _OHP * Cout * _OWP * 4,
    )
    out2 = pl.pallas_call(
        functools.partial(_conv_gelu_body, lb=LB),
        out_shape=jax.ShapeDtypeStruct((N, _OHP, Cout * _OWP), jnp.float32),
        grid=(N,),
        in_specs=[
            pl.BlockSpec((1, _UP, _NPH * LB), lambda n: (n, 0, 0)),
            pl.BlockSpec((_NPH * _NTAP, LB, Cout * _OWP), lambda n: (0, 0, 0)),
            pl.BlockSpec((1, Cout * _OWP), lambda n: (0, 0)),
        ],
        out_specs=pl.BlockSpec((1, _OHP, Cout * _OWP), lambda n: (n, 0, 0)),
        compiler_params=pltpu.CompilerParams(
            dimension_semantics=("parallel",),
            vmem_limit_bytes=48 * 1024 * 1024),
        cost_estimate=cost,
    )(xs, g, bvec)

    out = out2.reshape(N, _OHP, Cout, _OWP)[:, :OH, :, :OW]
    return out.transpose(0, 2, 1, 3)
